# baseline (speedup 1.0000x reference)
import functools

import jax
import jax.numpy as jnp
import numpy as np
from jax import lax
from jax.experimental import pallas as pl
from jax.experimental.pallas import tpu as pltpu
from jax.experimental.pallas import tpu_sc as plsc

_NUM_WORKERS = 32
_CHUNK = 256
_K0 = 128
_BN = 4096
_SH = _BN.bit_length() - 1

def _tc_project_pack(tT, tail, ws2):
  H, V = tT.shape
  nblk = -(-V // (4 * _BN))
  vtail = V - (nblk - 1) * 4 * _BN
  hw = H // 2

  def body(t_ref, tail_ref, w_ref, o_ref):
    i = pl.program_id(0)

    def quarter(blk):
      t2 = lax.dot_general(blk, w_ref[...], (((0,), (0,)), ((), ())),
                           precision=lax.Precision.HIGHEST,
                           preferred_element_type=jnp.float32)
      halves = []
      for h in range(2):
        lo = t2[:, 32 * h:32 * h + 16]
        hi = t2[:, 32 * h + 16:32 * h + 32]
        lo_b = lax.bitcast_convert_type(
            lo.astype(jnp.bfloat16), jnp.uint16).astype(jnp.int32)
        hi_b = lax.bitcast_convert_type(
            hi.astype(jnp.bfloat16), jnp.uint16).astype(jnp.int32)
        halves.append(
            lax.bitcast_convert_type(lo_b | (hi_b << 16), jnp.float32))
      return jnp.concatenate(halves, axis=1)

    for q in range(4):
      o_ref[:, q * hw:(q + 1) * hw] = quarter(
          t_ref[:, q * _BN:(q + 1) * _BN])

    @pl.when(i == nblk - 1)
    def _():
      o_ref[0:vtail, 0:hw] = quarter(tail_ref[...])[_BN - vtail:_BN, :]

  return pl.pallas_call(
      body,
      grid=(nblk,),
      in_specs=[
          pl.BlockSpec((H, 4 * _BN), lambda i: (0, i)),
          pl.BlockSpec((H, _BN), lambda i: (0, 0)),
          pl.BlockSpec((H, H), lambda i: (0, 0)),
      ],
      out_specs=pl.BlockSpec((_BN, 2 * H), lambda i: (i, 0)),
      out_shape=jax.ShapeDtypeStruct((nblk * _BN, 2 * H), jnp.float32),
  )(tT, tail, ws2)


def _sc_gather_pool(x, t_lin, bias):
  B, L = x.shape
  _, HW = t_lin.shape
  H = 2 * HW
  bpw = B // _NUM_WORKERS
  n_chunks = bpw // _CHUNK
  k1 = L - _K0
  nw = HW // 16
  nc = H // 16
  nv = -(-L // 16)
  mesh = plsc.VectorSubcoreMesh(core_axis_name="c", subcore_axis_name="s")

  @functools.partial(
      pl.kernel,
      mesh=mesh,
      compiler_params=pltpu.CompilerParams(use_tc_tiling_on_sc=False,
                                           needs_layout_passes=False),
      out_type=jax.ShapeDtypeStruct((B, H), jnp.float32),
      scratch_types=[
          pltpu.VMEM((_CHUNK, L), jnp.int32),
          pltpu.VMEM((2, L), jnp.int32),
          pltpu.VMEM((2, L, HW), jnp.float32),
          pltpu.VMEM((_CHUNK, H), jnp.float32),
          pltpu.VMEM((H,), jnp.float32),
          pltpu.SemaphoreType.DMA,
          pltpu.SemaphoreType.DMA,
      ],
  )
  def sc_kernel(x_hbm, tab_hbm, b_hbm, out_hbm,
                idx_v, idx2_v, rows_v, acc_v, b_v, sem0, sem1):
    wid = lax.axis_index("s") * 2 + lax.axis_index("c")
    base = wid * bpw
    sems = (sem0, sem1)
    pltpu.sync_copy(b_hbm, b_v)
    bias_r = [b_v[pl.ds(c * 16, 16)] for c in range(nc)]
    himask = jnp.int32(-65536)

    def fire(g, slot):
      for v in range(nv):
        s = min(v * 16, L - 16)
        r = idx_v[g, pl.ds(s, 16)]
        j = ((r & ~jnp.int32(4 * _BN - 1))
             | ((r & jnp.int32(_BN - 1)) << 2)
             | ((r >> _SH) & jnp.int32(3)))
        idx2_v[slot, pl.ds(s, 16)] = j
      pltpu.async_copy(tab_hbm.at[idx2_v.at[slot, pl.ds(0, _K0)]],
                       rows_v.at[slot, pl.ds(0, _K0)], sems[slot])
      pltpu.async_copy(tab_hbm.at[idx2_v.at[slot, pl.ds(_K0, k1)]],
                       rows_v.at[slot, pl.ds(_K0, k1)], sems[slot])

    def drain(slot):
      pltpu.make_async_copy(tab_hbm.at[idx2_v.at[slot, pl.ds(0, _K0)]],
                            rows_v.at[slot, pl.ds(0, _K0)], sems[slot]).wait()
      pltpu.make_async_copy(tab_hbm.at[idx2_v.at[slot, pl.ds(_K0, k1)]],
                            rows_v.at[slot, pl.ds(_K0, k1)], sems[slot]).wait()

    for ch in range(n_chunks):
      cbase = base + ch * _CHUNK
      pltpu.sync_copy(x_hbm.at[pl.ds(cbase, _CHUNK)], idx_v)
      fire(0, 0)
      fire(1, 1)

      @pl.loop(0, _CHUNK, step=2)
      def _pair(g2):
        for slot in range(2):
          g = g2 + slot
          drain(slot)

          def body(r, accs):
            accs = list(accs)
            for w in range(nw):
              word = plsc.bitcast(rows_v[slot, r, pl.ds(w * 16, 16)],
                                  jnp.int32)
              lo = plsc.bitcast(word << 16, jnp.float32)
              hi = plsc.bitcast(word & himask, jnp.float32)
              accs[2 * w] = accs[2 * w] + lo
              accs[2 * w + 1] = accs[2 * w + 1] + hi
            return tuple(accs)

          accs = lax.fori_loop(0, L, body, tuple(bias_r), unroll=8)
          for c in range(nc):
            acc_v[g, pl.ds(c * 16, 16)] = accs[c]

          @pl.when(g + 2 < _CHUNK)
          def _():
            fire(g + 2, slot)

      pltpu.sync_copy(acc_v, out_hbm.at[pl.ds(cbase, _CHUNK)])

  return sc_kernel(x, t_lin, bias)


def kernel(x, table, W, b):
  B, L = x.shape
  V, H = table.shape
  tT = table.T
  tail = lax.slice(tT, (0, V - _BN), (H, V))
  ws2 = W.T * (1.0 / L)
  packed = _tc_project_pack(tT, tail, ws2)
  t_lin = packed.reshape(-1, H // 2)
  return _sc_gather_pool(x.astype(jnp.int32), t_lin, b)


# speedup vs baseline: 1.4819x; 1.4819x over previous
import functools

import jax
import jax.numpy as jnp
import numpy as np
from jax import lax
from jax.experimental import pallas as pl
from jax.experimental.pallas import tpu as pltpu
from jax.experimental.pallas import tpu_sc as plsc

_NUM_WORKERS = 32
_CHUNK = 256
_K0 = 128
_BN = 4096
_SH = _BN.bit_length() - 1

def _tc_project_pack(tT, tail, ws2):
  H, V = tT.shape
  nblk = -(-V // (4 * _BN))
  vtail = V - (nblk - 1) * 4 * _BN
  hw = H // 2

  def body(t_ref, tail_ref, w_ref, o_ref):
    i = pl.program_id(0)

    def quarter(blk):
      t2 = lax.dot_general(blk, w_ref[...], (((0,), (0,)), ((), ())),
                           preferred_element_type=jnp.float32)
      lo_b = lax.bitcast_convert_type(
          t2[:, 0:32].astype(jnp.bfloat16), jnp.uint16).astype(jnp.int32)
      hi_b = lax.bitcast_convert_type(
          t2[:, 32:64].astype(jnp.bfloat16), jnp.uint16).astype(jnp.int32)
      return lax.bitcast_convert_type(lo_b | (hi_b << 16), jnp.float32)

    for q in range(4):
      o_ref[:, q * hw:(q + 1) * hw] = quarter(
          t_ref[:, q * _BN:(q + 1) * _BN])

    @pl.when(i == nblk - 1)
    def _():
      o_ref[0:vtail, 0:hw] = quarter(tail_ref[...])[_BN - vtail:_BN, :]

  return pl.pallas_call(
      body,
      grid=(nblk,),
      in_specs=[
          pl.BlockSpec((H, 4 * _BN), lambda i: (0, i)),
          pl.BlockSpec((H, _BN), lambda i: (0, 0)),
          pl.BlockSpec((H, H), lambda i: (0, 0)),
      ],
      out_specs=pl.BlockSpec((_BN, 2 * H), lambda i: (i, 0)),
      out_shape=jax.ShapeDtypeStruct((nblk * _BN, 2 * H), jnp.float32),
  )(tT, tail, ws2)


def _sc_gather_pool(x, t_lin, bias):
  B, L = x.shape
  _, HW = t_lin.shape
  H = 2 * HW
  bpw = B // _NUM_WORKERS
  n_chunks = bpw // _CHUNK
  k1 = L - _K0
  nw = HW // 16
  nc = H // 16
  nv = -(-L // 16)
  mesh = plsc.VectorSubcoreMesh(core_axis_name="c", subcore_axis_name="s")

  @functools.partial(
      pl.kernel,
      mesh=mesh,
      compiler_params=pltpu.CompilerParams(use_tc_tiling_on_sc=False,
                                           needs_layout_passes=False),
      out_type=jax.ShapeDtypeStruct((B, H), jnp.float32),
      scratch_types=[
          pltpu.VMEM((_CHUNK, L), jnp.int32),
          pltpu.VMEM((2, L), jnp.int32),
          pltpu.VMEM((2, L, HW), jnp.float32),
          pltpu.VMEM((_CHUNK, H), jnp.float32),
          pltpu.VMEM((H,), jnp.float32),
          pltpu.SemaphoreType.DMA,
          pltpu.SemaphoreType.DMA,
      ],
  )
  def sc_kernel(x_hbm, tab_hbm, b_hbm, out_hbm,
                idx_v, idx2_v, rows_v, acc_v, b_v, sem0, sem1):
    wid = lax.axis_index("s") * 2 + lax.axis_index("c")
    base = wid * bpw
    sems = (sem0, sem1)
    pltpu.sync_copy(b_hbm, b_v)
    bias_r = [b_v[pl.ds(c * 16, 16)] for c in range(nc)]
    himask = jnp.int32(-65536)

    def fire(g, slot):
      for v in range(nv):
        s = min(v * 16, L - 16)
        r = idx_v[g, pl.ds(s, 16)]
        j = ((r & ~jnp.int32(4 * _BN - 1))
             | ((r & jnp.int32(_BN - 1)) << 2)
             | ((r >> _SH) & jnp.int32(3)))
        idx2_v[slot, pl.ds(s, 16)] = j
      pltpu.async_copy(tab_hbm.at[idx2_v.at[slot, pl.ds(0, _K0)]],
                       rows_v.at[slot, pl.ds(0, _K0)], sems[slot])
      pltpu.async_copy(tab_hbm.at[idx2_v.at[slot, pl.ds(_K0, k1)]],
                       rows_v.at[slot, pl.ds(_K0, k1)], sems[slot])

    def drain(slot):
      pltpu.make_async_copy(tab_hbm.at[idx2_v.at[slot, pl.ds(0, _K0)]],
                            rows_v.at[slot, pl.ds(0, _K0)], sems[slot]).wait()
      pltpu.make_async_copy(tab_hbm.at[idx2_v.at[slot, pl.ds(_K0, k1)]],
                            rows_v.at[slot, pl.ds(_K0, k1)], sems[slot]).wait()

    for ch in range(n_chunks):
      cbase = base + ch * _CHUNK
      pltpu.sync_copy(x_hbm.at[pl.ds(cbase, _CHUNK)], idx_v)
      fire(0, 0)
      fire(1, 1)

      @pl.loop(0, _CHUNK, step=2)
      def _pair(g2):
        for slot in range(2):
          g = g2 + slot
          drain(slot)

          def body(r, accs):
            accs = list(accs)
            for w in range(nw):
              word = plsc.bitcast(rows_v[slot, r, pl.ds(w * 16, 16)],
                                  jnp.int32)
              lo = plsc.bitcast(word << 16, jnp.float32)
              hi = plsc.bitcast(word & himask, jnp.float32)
              accs[w] = accs[w] + lo
              accs[2 + w] = accs[2 + w] + hi
            return tuple(accs)

          accs = lax.fori_loop(0, L, body, tuple(bias_r), unroll=8)
          for c in range(nc):
            acc_v[g, pl.ds(c * 16, 16)] = accs[c]

          @pl.when(g + 2 < _CHUNK)
          def _():
            fire(g + 2, slot)

      pltpu.sync_copy(acc_v, out_hbm.at[pl.ds(cbase, _CHUNK)])

  return sc_kernel(x, t_lin, bias)


def kernel(x, table, W, b):
  B, L = x.shape
  V, H = table.shape
  tT = table.T
  tail = lax.slice(tT, (0, V - _BN), (H, V))
  ws2 = W.T * (1.0 / L)
  packed = _tc_project_pack(tT, tail, ws2)
  t_lin = packed.reshape(-1, H // 2)
  return _sc_gather_pool(x.astype(jnp.int32), t_lin, b)


# speedup vs baseline: 1.5266x; 1.0302x over previous
import functools

import jax
import jax.numpy as jnp
import numpy as np
from jax import lax
from jax.experimental import pallas as pl
from jax.experimental.pallas import tpu as pltpu
from jax.experimental.pallas import tpu_sc as plsc

_NUM_WORKERS = 32
_CHUNK = 256
_K0 = 128
_BN = 4096
_SH = _BN.bit_length() - 1

def _tc_project_pack(tT, tail, ws2):
  H, V = tT.shape
  nblk = -(-V // (4 * _BN))
  vtail = V - (nblk - 1) * 4 * _BN
  hw = H // 2

  def body(t_ref, tail_ref, w_ref, o_ref):
    i = pl.program_id(0)

    def quarter(blk):
      t2 = lax.dot_general(blk, w_ref[...], (((0,), (0,)), ((), ())),
                           preferred_element_type=jnp.float32)
      lo_b = lax.bitcast_convert_type(
          t2[:, 0:32].astype(jnp.bfloat16), jnp.uint16).astype(jnp.int32)
      hi_b = lax.bitcast_convert_type(
          t2[:, 32:64].astype(jnp.bfloat16), jnp.uint16).astype(jnp.int32)
      return lax.bitcast_convert_type(lo_b | (hi_b << 16), jnp.float32)

    for q in range(4):
      o_ref[:, q * hw:(q + 1) * hw] = quarter(
          t_ref[:, q * _BN:(q + 1) * _BN])

    @pl.when(i == nblk - 1)
    def _():
      o_ref[0:vtail, 0:hw] = quarter(tail_ref[...])[_BN - vtail:_BN, :]

  return pl.pallas_call(
      body,
      grid=(nblk,),
      in_specs=[
          pl.BlockSpec((H, 4 * _BN), lambda i: (0, i)),
          pl.BlockSpec((H, _BN), lambda i: (0, 0)),
          pl.BlockSpec((H, H), lambda i: (0, 0)),
      ],
      out_specs=pl.BlockSpec((_BN, 2 * H), lambda i: (i, 0)),
      out_shape=jax.ShapeDtypeStruct((nblk * _BN, 2 * H), jnp.float32),
  )(tT, tail, ws2)


def _sc_gather_pool(x, t_lin, bias):
  B, L = x.shape
  _, HW = t_lin.shape
  H = 2 * HW
  bpw = B // _NUM_WORKERS
  n_chunks = bpw // _CHUNK
  k1 = L - _K0
  nw = HW // 16
  nc = H // 16
  nv = -(-L // 16)
  mesh = plsc.VectorSubcoreMesh(core_axis_name="c", subcore_axis_name="s")

  @functools.partial(
      pl.kernel,
      mesh=mesh,
      compiler_params=pltpu.CompilerParams(use_tc_tiling_on_sc=False,
                                           needs_layout_passes=False),
      out_type=jax.ShapeDtypeStruct((B, H), jnp.float32),
      scratch_types=[
          pltpu.VMEM((_CHUNK, L), jnp.int32),
          pltpu.VMEM((2, L), jnp.int32),
          pltpu.VMEM((2, L, HW), jnp.float32),
          pltpu.VMEM((_CHUNK, H), jnp.float32),
          pltpu.VMEM((H,), jnp.float32),
          pltpu.SemaphoreType.DMA,
          pltpu.SemaphoreType.DMA,
      ],
  )
  def sc_kernel(x_hbm, tab_hbm, b_hbm, out_hbm,
                idx_v, idx2_v, rows_v, acc_v, b_v, sem0, sem1):
    wid = lax.axis_index("s") * 2 + lax.axis_index("c")
    base = wid * bpw
    sems = (sem0, sem1)
    pltpu.sync_copy(b_hbm, b_v)
    bias_r = [b_v[pl.ds(c * 16, 16)] for c in range(nc)]
    himask = jnp.int32(-65536)

    def fire(g, slot):
      for v in range(nv):
        s = min(v * 16, L - 16)
        r = idx_v[g, pl.ds(s, 16)]
        j = ((r & ~jnp.int32(4 * _BN - 1))
             | ((r & jnp.int32(_BN - 1)) << 2)
             | ((r >> _SH) & jnp.int32(3)))
        idx2_v[slot, pl.ds(s, 16)] = j
      pltpu.async_copy(tab_hbm.at[idx2_v.at[slot, pl.ds(0, _K0)]],
                       rows_v.at[slot, pl.ds(0, _K0)], sems[slot])
      pltpu.async_copy(tab_hbm.at[idx2_v.at[slot, pl.ds(_K0, k1)]],
                       rows_v.at[slot, pl.ds(_K0, k1)], sems[slot])

    def drain(slot):
      pltpu.make_async_copy(tab_hbm.at[idx2_v.at[slot, pl.ds(0, _K0)]],
                            rows_v.at[slot, pl.ds(0, _K0)], sems[slot]).wait()
      pltpu.make_async_copy(tab_hbm.at[idx2_v.at[slot, pl.ds(_K0, k1)]],
                            rows_v.at[slot, pl.ds(_K0, k1)], sems[slot]).wait()

    for ch in range(n_chunks):
      cbase = base + ch * _CHUNK
      pltpu.sync_copy(x_hbm.at[pl.ds(cbase, _CHUNK)], idx_v)
      fire(0, 0)
      fire(1, 1)

      @pl.loop(0, _CHUNK, step=2)
      def _pair(g2):
        for slot in range(2):
          g = g2 + slot
          drain(slot)

          def body(gi, accs):
            rbase = gi * 8
            zb = jnp.zeros((2 * 16,), jnp.bfloat16)
            grp = [zb] * nw
            for t in range(8):
              for w in range(nw):
                grp[w] = grp[w] + plsc.bitcast(
                    rows_v[slot, rbase + t, pl.ds(w * 16, 16)],
                    jnp.bfloat16)
            accs = list(accs)
            for w in range(nw):
              word = plsc.bitcast(grp[w], jnp.int32)
              accs[w] = accs[w] + plsc.bitcast(word << 16, jnp.float32)
              accs[2 + w] = accs[2 + w] + plsc.bitcast(word & himask,
                                                       jnp.float32)
            return tuple(accs)

          accs = lax.fori_loop(0, L // 8, body, tuple(bias_r), unroll=2)
          for c in range(nc):
            acc_v[g, pl.ds(c * 16, 16)] = accs[c]

          @pl.when(g + 2 < _CHUNK)
          def _():
            fire(g + 2, slot)

      pltpu.sync_copy(acc_v, out_hbm.at[pl.ds(cbase, _CHUNK)])

  return sc_kernel(x, t_lin, bias)


def kernel(x, table, W, b):
  B, L = x.shape
  V, H = table.shape
  tT = table.T
  tail = lax.slice(tT, (0, V - _BN), (H, V))
  ws2 = W.T * (1.0 / L)
  packed = _tc_project_pack(tT, tail, ws2)
  t_lin = packed.reshape(-1, H // 2)
  return _sc_gather_pool(x.astype(jnp.int32), t_lin, b)
